# baseline (speedup 1.0000x reference)
import jax
import jax.numpy as jnp
from jax.experimental import pallas as pl
from jax.experimental.pallas import tpu as pltpu

N_HEAD = 16
DH = 64
D = 1024
DI = 4096
T = 256
M = 64
LM = 128
NCHUNK = 8
CI = DI // NCHUNK


def _ln(x, g, b, eps=1e-5):
    m = jnp.mean(x, axis=-1, keepdims=True)
    v = jnp.mean((x - m) ** 2, axis=-1, keepdims=True)
    return (x - m) / jnp.sqrt(v + eps) * g + b


def _dot(a, b, dims):
    return jax.lax.dot_general(a, b, dimension_numbers=(dims, ((), ())),
                               preferred_element_type=jnp.float32)


NG = 96
GS = 8
SL = NG * GS


def _prep_body(dec_ref, mem_ref, memT_ref, g0_ref, be0_ref, Wq_ref, bq_ref,
               x_ref, qpad_ref, valid_ref, gent_ref, P2_ref):
    x = _ln(dec_ref[...], g0_ref[...], be0_ref[...])
    x_ref[...] = x
    q = jnp.dot(x, Wq_ref[...],
                preferred_element_type=jnp.float32) + bq_ref[...]

    v = mem_ref[...]
    mx = jnp.max(v, axis=1, keepdims=True)
    io = jax.lax.broadcasted_iota(jnp.int32, v.shape, 1)
    am_col = jnp.min(jnp.where(v == mx, io, M + 1), axis=1, keepdims=True)
    s_col = jnp.maximum(am_col - 1, 0)
    valid_ref[...] = (am_col != 0).astype(jnp.float32)

    vT = memT_ref[...]
    mxT = jnp.max(vT, axis=0, keepdims=True)
    ioT = jax.lax.broadcasted_iota(jnp.int32, vT.shape, 0)
    am_row = jnp.min(jnp.where(vT == mxT, ioT, M + 1), axis=0, keepdims=True)
    s_row = jnp.maximum(am_row - 1, 0)

    i32 = jnp.int32

    t_col = jax.lax.broadcasted_iota(i32, (T, T), 0)
    t_row = jax.lax.broadcasted_iota(i32, (T, T), 1)
    lt = (s_row < s_col) | ((s_row == s_col) & (t_row < t_col))
    rank = jnp.sum(lt.astype(i32), axis=1, keepdims=True)
    ltT = (s_col < s_row) | ((s_col == s_row) & (t_col < t_row))
    rankT = jnp.sum(ltT.astype(i32), axis=0, keepdims=True)

    e_col = jax.lax.broadcasted_iota(i32, (M, T), 0)
    cnt_col = jnp.sum((s_row == e_col).astype(i32), axis=1, keepdims=True)
    cumx_col = jnp.sum((s_row < e_col).astype(i32), axis=1, keepdims=True)
    e_rowM = jax.lax.broadcasted_iota(i32, (T, M), 1)
    cnt_row = jnp.sum((s_col == e_rowM).astype(i32), axis=0, keepdims=True)
    cumx_row = jnp.sum((s_col < e_rowM).astype(i32), axis=0, keepdims=True)
    gpe_col = (cnt_col + (GS - 1)) // GS
    gpe_row = (cnt_row + (GS - 1)) // GS

    eA = jax.lax.broadcasted_iota(i32, (M, M), 0)
    eB = jax.lax.broadcasted_iota(i32, (M, M), 1)
    gcumx_col = jnp.sum(jnp.where(eB < eA, gpe_row, 0), axis=1,
                        keepdims=True)
    gcumi_col = gcumx_col + gpe_col
    gcumx_row = jnp.sum(jnp.where(eA < eB, gpe_col, 0), axis=0,
                        keepdims=True)
    gcumi_row = gcumx_row + gpe_row

    gR96 = jax.lax.broadcasted_iota(i32, (M, NG), 1)
    e_of_g = jnp.sum((gcumi_col <= gR96).astype(i32), axis=0, keepdims=True)
    gent_ref[...] = jnp.minimum(e_of_g, M - 1)

    iR = jax.lax.broadcasted_iota(i32, (1, SL), 1)
    gRs = iR // GS
    rrR = iR % GS
    gcumi_colS = gcumi_col
    e_colS = jax.lax.broadcasted_iota(i32, (M, SL), 0)
    e_ofR = jnp.minimum(
        jnp.sum((gcumi_colS <= gRs).astype(i32), axis=0, keepdims=True),
        M - 1)
    ohR = (e_colS == e_ofR)
    gcumx_ofR = jnp.sum(jnp.where(ohR, gcumx_col, 0), axis=0, keepdims=True)
    cumx_ofR = jnp.sum(jnp.where(ohR, cumx_col, 0), axis=0, keepdims=True)
    cnt_ofR = jnp.sum(jnp.where(ohR, cnt_col, 0), axis=0, keepdims=True)
    kR = gRs - gcumx_ofR
    posR = cumx_ofR + GS * kR + rrR
    realR = (GS * kR + rrR) < cnt_ofR

    iC = jax.lax.broadcasted_iota(i32, (SL, 1), 0)
    gCs = iC // GS
    rrC = iC % GS
    e_rowS = jax.lax.broadcasted_iota(i32, (SL, M), 1)
    e_ofC = jnp.minimum(
        jnp.sum((gcumi_row <= gCs).astype(i32), axis=1, keepdims=True),
        M - 1)
    ohC = (e_rowS == e_ofC)
    gcumx_ofC = jnp.sum(jnp.where(ohC, gcumx_row, 0), axis=1, keepdims=True)
    cumx_ofC = jnp.sum(jnp.where(ohC, cumx_row, 0), axis=1, keepdims=True)
    kC = gCs - gcumx_ofC
    posC = cumx_ofC + GS * kC + rrC

    P = (posC == rankT).astype(jnp.float32)
    qpad_ref[...] = jnp.dot(P, q, preferred_element_type=jnp.float32)
    P2_ref[...] = ((rank == posR) & realR).astype(jnp.float32)


def _prep(dec2, mem2, mem2T, g0, be0, Wq, bq):
    return pl.pallas_call(
        _prep_body,
        out_shape=(
            jax.ShapeDtypeStruct((T, D), jnp.float32),
            jax.ShapeDtypeStruct((SL, D), jnp.float32),
            jax.ShapeDtypeStruct((T, 1), jnp.float32),
            jax.ShapeDtypeStruct((1, NG), jnp.int32),
            jax.ShapeDtypeStruct((T, SL), jnp.float32),
        ),
    )(dec2, mem2, mem2T, g0, be0, Wq, bq)


def _mfold_body(q_ref, WkR_ref, M_ref):
    qh = q_ref[0]
    Wkh = WkR_ref[0]
    Mh = _dot(qh, Wkh, ((1,), (1,)))
    M_ref[...] = Mh.reshape(SL, 1, D // 128, 128)


def _mfold(qpadT, WkR):
    return pl.pallas_call(
        _mfold_body,
        grid=(N_HEAD,),
        in_specs=[
            pl.BlockSpec((1, SL, DH), lambda h: (h, 0, 0)),
            pl.BlockSpec((1, D, DH), lambda h: (h, 0, 0)),
        ],
        out_specs=pl.BlockSpec((SL, 1, D // 128, 128),
                               lambda h: (0, h, 0, 0)),
        out_shape=jax.ShapeDtypeStruct((SL, N_HEAD, D // 128, 128),
                                       jnp.float32),
    )(qpadT, WkR)


GH = GS * N_HEAD


def _attend_body(gent_ref, M_ref, enc_ref, tgt_ref, bias_ref, ctx_ref):
    Mg = M_ref[0]
    enc = enc_ref[0]
    scores = _dot(Mg, enc, ((1,), (1,))) * 0.125 + bias_ref[0]
    mx = jnp.max(scores, axis=1, keepdims=True)
    e = jnp.exp(scores - mx)
    attn = e / jnp.sum(e, axis=1, keepdims=True)
    ctx_ref[0] = _dot(attn, tgt_ref[0], ((1,), (0,)))


def _attend(gent, Mg3, enc_mem, tgt_mem, biasT):
    grid_spec = pltpu.PrefetchScalarGridSpec(
        num_scalar_prefetch=1,
        grid=(NG,),
        in_specs=[
            pl.BlockSpec((1, GH, D), lambda i, g: (i, 0, 0)),
            pl.BlockSpec((1, LM, D), lambda i, g: (g[i], 0, 0)),
            pl.BlockSpec((1, LM, D), lambda i, g: (g[i], 0, 0)),
            pl.BlockSpec((1, 1, LM), lambda i, g: (g[i], 0, 0)),
        ],
        out_specs=pl.BlockSpec((1, GH, D), lambda i, g: (i, 0, 0)),
    )
    return pl.pallas_call(
        _attend_body,
        grid_spec=grid_spec,
        out_shape=jax.ShapeDtypeStruct((NG, GH, D), jnp.float32),
    )(gent, Mg3, enc_mem, tgt_mem, biasT)


def _tail_body(ctx_ref, P2_ref, Wv_ref, bv_ref, WoR_ref, x_ref, bo_ref,
               valid_ref, g1_ref, be1_ref, W1a_ref, b1a_ref, W2a_ref,
               b2a_ref, W1b_ref, b1b_ref, W2b_ref, b2b_ref, out_ref,
               accp_s, acc_s, u_s, z_s):
    g = pl.program_id(0)

    @pl.when(g == 0)
    def _():
        accp_s[...] = jnp.zeros_like(accp_s)

    @pl.when(g < N_HEAD)
    def _():
        ctx_h = ctx_ref[...].reshape(SL, D)
        st_h = _dot(ctx_h, Wv_ref[0], ((1,), (0,))) + bv_ref[0]
        accp_s[...] += _dot(st_h, WoR_ref[0], ((1,), (0,)))

    @pl.when(g == N_HEAD - 1)
    def _():
        u_s[...] = (jnp.dot(P2_ref[...], accp_s[...],
                            preferred_element_type=jnp.float32)
                    + bo_ref[...] + x_ref[...])
        acc_s[...] = jnp.zeros_like(acc_s)

    @pl.when((g >= N_HEAD) & (g < N_HEAD + NCHUNK))
    def _():
        h = jnp.maximum(
            jnp.dot(u_s[...], W1a_ref[...],
                    preferred_element_type=jnp.float32) + b1a_ref[...], 0.0)
        acc_s[...] += jnp.dot(h, W2a_ref[...],
                              preferred_element_type=jnp.float32)

    @pl.when(g == N_HEAD + NCHUNK - 1)
    def _():
        v = acc_s[...] + b2a_ref[...] + u_s[...]
        w = _ln(v, g1_ref[...], be1_ref[...]) * valid_ref[...]
        z_s[...] = x_ref[...] + w
        acc_s[...] = jnp.zeros_like(acc_s)

    @pl.when(g >= N_HEAD + NCHUNK)
    def _():
        h = jnp.maximum(
            jnp.dot(z_s[...], W1b_ref[...],
                    preferred_element_type=jnp.float32) + b1b_ref[...], 0.0)
        acc_s[...] += jnp.dot(h, W2b_ref[...],
                              preferred_element_type=jnp.float32)

    @pl.when(g == N_HEAD + 2 * NCHUNK - 1)
    def _():
        out_ref[...] = acc_s[...] + b2b_ref[...] + z_s[...]


def _tail(ctx4, P2, Wv4, bv4, WoR, x, bo, valid, g1, be1,
          W1a, b1a, W2a, b2a, W1b, b1b, W2b, b2b):
    hidx = lambda g: jnp.minimum(g, N_HEAD - 1)
    aidx = lambda g: jnp.clip(g - N_HEAD, 0, NCHUNK - 1)
    bidx = lambda g: jnp.clip(g - N_HEAD - NCHUNK, 0, NCHUNK - 1)
    return pl.pallas_call(
        _tail_body,
        grid=(N_HEAD + 2 * NCHUNK,),
        in_specs=[
            pl.BlockSpec((SL, 1, D // 128, 128),
                         lambda g: (0, hidx(g), 0, 0)),
            pl.BlockSpec((T, SL), lambda g: (0, 0)),
            pl.BlockSpec((1, D, DH), lambda g: (hidx(g), 0, 0)),
            pl.BlockSpec((1, 1, DH), lambda g: (hidx(g), 0, 0)),
            pl.BlockSpec((1, DH, D), lambda g: (hidx(g), 0, 0)),
            pl.BlockSpec((T, D), lambda g: (0, 0)),
            pl.BlockSpec((1, D), lambda g: (0, 0)),
            pl.BlockSpec((T, 1), lambda g: (0, 0)),
            pl.BlockSpec((1, D), lambda g: (0, 0)),
            pl.BlockSpec((1, D), lambda g: (0, 0)),
            pl.BlockSpec((D, CI), lambda g: (0, aidx(g))),
            pl.BlockSpec((1, CI), lambda g: (0, aidx(g))),
            pl.BlockSpec((CI, D), lambda g: (aidx(g), 0)),
            pl.BlockSpec((1, D), lambda g: (0, 0)),
            pl.BlockSpec((D, CI), lambda g: (0, bidx(g))),
            pl.BlockSpec((1, CI), lambda g: (0, bidx(g))),
            pl.BlockSpec((CI, D), lambda g: (bidx(g), 0)),
            pl.BlockSpec((1, D), lambda g: (0, 0)),
        ],
        out_specs=pl.BlockSpec((T, D), lambda g: (0, 0)),
        out_shape=jax.ShapeDtypeStruct((T, D), jnp.float32),
        scratch_shapes=[
            pltpu.VMEM((SL, D), jnp.float32),
            pltpu.VMEM((T, D), jnp.float32),
            pltpu.VMEM((T, D), jnp.float32),
            pltpu.VMEM((T, D), jnp.float32),
        ],
    )(ctx4, P2, Wv4, bv4, WoR, x, bo, valid, g1, be1,
      W1a, b1a, W2a, b2a, W1b, b1b, W2b, b2b)


def kernel(dec_output, tgt_mask, mem_attn_out, enc_out_mem, tgt_emb_mem,
           tgt_mask_mem, Wq, bq, Wk, bk, Wv, bv, Wo, bo, g0, be0, g1, be1,
           W1a, b1a, W2a, b2a, W1b, b1b, W2b, b2b):
    b, l_tar, d = dec_output.shape

    dec2 = dec_output.reshape(T, D)
    mem2 = mem_attn_out.reshape(T, M + 1)
    mem2T = mem2.T
    row = lambda v: v.reshape(1, -1)

    x, qpad, valid, gent, P2 = _prep(dec2, mem2, mem2T, row(g0), row(be0),
                                     Wq, row(bq))

    qpadT = qpad.reshape(SL, N_HEAD, DH).transpose(1, 0, 2)
    WkR = Wk.reshape(D, N_HEAD, DH).transpose(1, 0, 2)
    Mpad = _mfold(qpadT, WkR)
    Mg3 = Mpad.reshape(NG, GH, D)

    biasT = jnp.where(tgt_mask_mem, 0.0, -1e9).astype(jnp.float32)[:, None, :]
    ctxg = _attend(gent.reshape(NG), Mg3, enc_out_mem, tgt_emb_mem, biasT)
    ctx4 = ctxg.reshape(SL, N_HEAD, D // 128, 128)

    WvR = Wv.reshape(D, N_HEAD, DH).transpose(1, 0, 2)
    WoR = Wo.reshape(N_HEAD, DH, D)
    out = _tail(ctx4, P2, WvR,
                bv.reshape(N_HEAD, 1, DH), WoR, x, row(bo),
                valid, row(g1), row(be1), W1a, row(b1a), W2a, row(b2a),
                W1b, row(b1b), W2b, row(b2b))
    return out.reshape(b, l_tar, d)


# speedup vs baseline: 1.3646x; 1.3646x over previous
import jax
import jax.numpy as jnp
from jax.experimental import pallas as pl
from jax.experimental.pallas import tpu as pltpu

N_HEAD = 16
DH = 64
D = 1024
DI = 4096
T = 256
M = 64
LM = 128
NCHUNK = 8
CI = DI // NCHUNK


def _ln(x, g, b, eps=1e-5):
    m = jnp.mean(x, axis=-1, keepdims=True)
    v = jnp.mean((x - m) ** 2, axis=-1, keepdims=True)
    return (x - m) / jnp.sqrt(v + eps) * g + b


def _dot(a, b, dims):
    return jax.lax.dot_general(a, b, dimension_numbers=(dims, ((), ())),
                               preferred_element_type=jnp.float32)


NG = 96
GS = 8
SL = NG * GS


def _prep_body(dec_ref, mem_ref, memT_ref, g0_ref, be0_ref, Wq_ref, bq_ref,
               x_ref, qpad_ref, valid_ref, gent_ref, P2_ref):
    x = _ln(dec_ref[...], g0_ref[...], be0_ref[...])
    x_ref[...] = x
    q = jnp.dot(x, Wq_ref[...],
                preferred_element_type=jnp.float32) + bq_ref[...]

    v = mem_ref[...]
    mx = jnp.max(v, axis=1, keepdims=True)
    io = jax.lax.broadcasted_iota(jnp.int32, v.shape, 1)
    am_col = jnp.min(jnp.where(v == mx, io, M + 1), axis=1, keepdims=True)
    s_col = jnp.maximum(am_col - 1, 0)
    valid_ref[...] = (am_col != 0).astype(jnp.float32)

    vT = memT_ref[...]
    mxT = jnp.max(vT, axis=0, keepdims=True)
    ioT = jax.lax.broadcasted_iota(jnp.int32, vT.shape, 0)
    am_row = jnp.min(jnp.where(vT == mxT, ioT, M + 1), axis=0, keepdims=True)
    s_row = jnp.maximum(am_row - 1, 0)

    i32 = jnp.int32

    t_col = jax.lax.broadcasted_iota(i32, (T, T), 0)
    t_row = jax.lax.broadcasted_iota(i32, (T, T), 1)
    lt = (s_row < s_col) | ((s_row == s_col) & (t_row < t_col))
    rank = jnp.sum(lt.astype(i32), axis=1, keepdims=True)
    ltT = (s_col < s_row) | ((s_col == s_row) & (t_col < t_row))
    rankT = jnp.sum(ltT.astype(i32), axis=0, keepdims=True)

    e_col = jax.lax.broadcasted_iota(i32, (M, T), 0)
    cnt_col = jnp.sum((s_row == e_col).astype(i32), axis=1, keepdims=True)
    cumx_col = jnp.sum((s_row < e_col).astype(i32), axis=1, keepdims=True)
    e_rowM = jax.lax.broadcasted_iota(i32, (T, M), 1)
    cnt_row = jnp.sum((s_col == e_rowM).astype(i32), axis=0, keepdims=True)
    cumx_row = jnp.sum((s_col < e_rowM).astype(i32), axis=0, keepdims=True)
    gpe_col = (cnt_col + (GS - 1)) // GS
    gpe_row = (cnt_row + (GS - 1)) // GS

    eA = jax.lax.broadcasted_iota(i32, (M, M), 0)
    eB = jax.lax.broadcasted_iota(i32, (M, M), 1)
    gcumx_col = jnp.sum(jnp.where(eB < eA, gpe_row, 0), axis=1,
                        keepdims=True)
    gcumi_col = gcumx_col + gpe_col
    gcumx_row = jnp.sum(jnp.where(eA < eB, gpe_col, 0), axis=0,
                        keepdims=True)
    gcumi_row = gcumx_row + gpe_row

    gR96 = jax.lax.broadcasted_iota(i32, (M, NG), 1)
    e_of_g = jnp.sum((gcumi_col <= gR96).astype(i32), axis=0, keepdims=True)
    gent_ref[...] = jnp.minimum(e_of_g, M - 1)

    iR = jax.lax.broadcasted_iota(i32, (1, SL), 1)
    gRs = iR // GS
    rrR = iR % GS
    gcumi_colS = gcumi_col
    e_colS = jax.lax.broadcasted_iota(i32, (M, SL), 0)
    e_ofR = jnp.minimum(
        jnp.sum((gcumi_colS <= gRs).astype(i32), axis=0, keepdims=True),
        M - 1)
    ohR = (e_colS == e_ofR)
    gcumx_ofR = jnp.sum(jnp.where(ohR, gcumx_col, 0), axis=0, keepdims=True)
    cumx_ofR = jnp.sum(jnp.where(ohR, cumx_col, 0), axis=0, keepdims=True)
    cnt_ofR = jnp.sum(jnp.where(ohR, cnt_col, 0), axis=0, keepdims=True)
    kR = gRs - gcumx_ofR
    posR = cumx_ofR + GS * kR + rrR
    realR = (GS * kR + rrR) < cnt_ofR

    iC = jax.lax.broadcasted_iota(i32, (SL, 1), 0)
    gCs = iC // GS
    rrC = iC % GS
    e_rowS = jax.lax.broadcasted_iota(i32, (SL, M), 1)
    e_ofC = jnp.minimum(
        jnp.sum((gcumi_row <= gCs).astype(i32), axis=1, keepdims=True),
        M - 1)
    ohC = (e_rowS == e_ofC)
    gcumx_ofC = jnp.sum(jnp.where(ohC, gcumx_row, 0), axis=1, keepdims=True)
    cumx_ofC = jnp.sum(jnp.where(ohC, cumx_row, 0), axis=1, keepdims=True)
    kC = gCs - gcumx_ofC
    posC = cumx_ofC + GS * kC + rrC

    P = (posC == rankT).astype(jnp.float32)
    qpad_ref[...] = jnp.dot(P, q, preferred_element_type=jnp.float32)
    P2_ref[...] = ((rank == posR) & realR).astype(jnp.float32)


def _prep(dec2, mem2, mem2T, g0, be0, Wq, bq):
    return pl.pallas_call(
        _prep_body,
        out_shape=(
            jax.ShapeDtypeStruct((T, D), jnp.float32),
            jax.ShapeDtypeStruct((SL, D), jnp.float32),
            jax.ShapeDtypeStruct((T, 1), jnp.float32),
            jax.ShapeDtypeStruct((1, NG), jnp.int32),
            jax.ShapeDtypeStruct((T, SL), jnp.float32),
        ),
    )(dec2, mem2, mem2T, g0, be0, Wq, bq)


def _mfold_body(q_ref, WkR_ref, M_ref):
    qh = q_ref[0]
    Wkh = WkR_ref[0]
    Mh = _dot(qh, Wkh, ((1,), (1,)))
    M_ref[...] = Mh.reshape(SL, 1, D // 128, 128)


def _mfold(qpadT, WkR):
    return pl.pallas_call(
        _mfold_body,
        grid=(N_HEAD,),
        in_specs=[
            pl.BlockSpec((1, SL, DH), lambda h: (h, 0, 0)),
            pl.BlockSpec((1, D, DH), lambda h: (h, 0, 0)),
        ],
        out_specs=pl.BlockSpec((SL, 1, D // 128, 128),
                               lambda h: (0, h, 0, 0)),
        out_shape=jax.ShapeDtypeStruct((SL, N_HEAD, D // 128, 128),
                                       jnp.float32),
    )(qpadT, WkR)


GH = GS * N_HEAD


def _attend_body(gent_ref, M_ref, enc_ref, tgt_ref, bias_ref, ctx_ref):
    Mg = M_ref[0].reshape(GH, D)
    enc = enc_ref[0]
    scores = _dot(Mg, enc, ((1,), (1,))) * 0.125 + bias_ref[0]
    mx = jnp.max(scores, axis=1, keepdims=True)
    e = jnp.exp(scores - mx)
    attn = e / jnp.sum(e, axis=1, keepdims=True)
    ctx = _dot(attn, tgt_ref[0], ((1,), (0,)))
    ctx_ref[0] = ctx.reshape(GH, D // 128, 128)


def _attend(gent, Mg5, enc_mem, tgt_mem, biasT):
    grid_spec = pltpu.PrefetchScalarGridSpec(
        num_scalar_prefetch=1,
        grid=(NG,),
        in_specs=[
            pl.BlockSpec((1, GH, D // 128, 128), lambda i, g: (i, 0, 0, 0)),
            pl.BlockSpec((1, LM, D), lambda i, g: (g[i], 0, 0)),
            pl.BlockSpec((1, LM, D), lambda i, g: (g[i], 0, 0)),
            pl.BlockSpec((1, 1, LM), lambda i, g: (g[i], 0, 0)),
        ],
        out_specs=pl.BlockSpec((1, GH, D // 128, 128),
                               lambda i, g: (i, 0, 0, 0)),
    )
    return pl.pallas_call(
        _attend_body,
        grid_spec=grid_spec,
        out_shape=jax.ShapeDtypeStruct((NG, GH, D // 128, 128),
                                       jnp.float32),
    )(gent, Mg5, enc_mem, tgt_mem, biasT)


def _tail_body(ctx_ref, P2_ref, Wv_ref, bv_ref, WoR_ref, x_ref, bo_ref,
               valid_ref, g1_ref, be1_ref, W1a_ref, b1a_ref, W2a_ref,
               b2a_ref, W1b_ref, b1b_ref, W2b_ref, b2b_ref, out_ref,
               accp_s, acc_s, u_s, z_s):
    g = pl.program_id(0)

    @pl.when(g == 0)
    def _():
        accp_s[...] = jnp.zeros_like(accp_s)

    @pl.when(g < N_HEAD)
    def _():
        ctx_h = ctx_ref[...].reshape(SL, D)
        st_h = _dot(ctx_h, Wv_ref[0], ((1,), (0,))) + bv_ref[0]
        accp_s[...] += _dot(st_h, WoR_ref[0], ((1,), (0,)))

    @pl.when(g == N_HEAD - 1)
    def _():
        u_s[...] = (jnp.dot(P2_ref[...], accp_s[...],
                            preferred_element_type=jnp.float32)
                    + bo_ref[...] + x_ref[...])
        acc_s[...] = jnp.zeros_like(acc_s)

    @pl.when((g >= N_HEAD) & (g < N_HEAD + NCHUNK))
    def _():
        h = jnp.maximum(
            jnp.dot(u_s[...], W1a_ref[...],
                    preferred_element_type=jnp.float32) + b1a_ref[...], 0.0)
        acc_s[...] += jnp.dot(h, W2a_ref[...],
                              preferred_element_type=jnp.float32)

    @pl.when(g == N_HEAD + NCHUNK - 1)
    def _():
        v = acc_s[...] + b2a_ref[...] + u_s[...]
        w = _ln(v, g1_ref[...], be1_ref[...]) * valid_ref[...]
        z_s[...] = x_ref[...] + w
        acc_s[...] = jnp.zeros_like(acc_s)

    @pl.when(g >= N_HEAD + NCHUNK)
    def _():
        h = jnp.maximum(
            jnp.dot(z_s[...], W1b_ref[...],
                    preferred_element_type=jnp.float32) + b1b_ref[...], 0.0)
        acc_s[...] += jnp.dot(h, W2b_ref[...],
                              preferred_element_type=jnp.float32)

    @pl.when(g == N_HEAD + 2 * NCHUNK - 1)
    def _():
        out_ref[...] = acc_s[...] + b2b_ref[...] + z_s[...]


def _tail(ctx4, P2, Wv4, bv4, WoR, x, bo, valid, g1, be1,
          W1a, b1a, W2a, b2a, W1b, b1b, W2b, b2b):
    hidx = lambda g: jnp.minimum(g, N_HEAD - 1)
    aidx = lambda g: jnp.clip(g - N_HEAD, 0, NCHUNK - 1)
    bidx = lambda g: jnp.clip(g - N_HEAD - NCHUNK, 0, NCHUNK - 1)
    return pl.pallas_call(
        _tail_body,
        grid=(N_HEAD + 2 * NCHUNK,),
        in_specs=[
            pl.BlockSpec((SL, 1, D // 128, 128),
                         lambda g: (0, hidx(g), 0, 0)),
            pl.BlockSpec((T, SL), lambda g: (0, 0)),
            pl.BlockSpec((1, D, DH), lambda g: (hidx(g), 0, 0)),
            pl.BlockSpec((1, 1, DH), lambda g: (hidx(g), 0, 0)),
            pl.BlockSpec((1, DH, D), lambda g: (hidx(g), 0, 0)),
            pl.BlockSpec((T, D), lambda g: (0, 0)),
            pl.BlockSpec((1, D), lambda g: (0, 0)),
            pl.BlockSpec((T, 1), lambda g: (0, 0)),
            pl.BlockSpec((1, D), lambda g: (0, 0)),
            pl.BlockSpec((1, D), lambda g: (0, 0)),
            pl.BlockSpec((D, CI), lambda g: (0, aidx(g))),
            pl.BlockSpec((1, CI), lambda g: (0, aidx(g))),
            pl.BlockSpec((CI, D), lambda g: (aidx(g), 0)),
            pl.BlockSpec((1, D), lambda g: (0, 0)),
            pl.BlockSpec((D, CI), lambda g: (0, bidx(g))),
            pl.BlockSpec((1, CI), lambda g: (0, bidx(g))),
            pl.BlockSpec((CI, D), lambda g: (bidx(g), 0)),
            pl.BlockSpec((1, D), lambda g: (0, 0)),
        ],
        out_specs=pl.BlockSpec((T, D), lambda g: (0, 0)),
        out_shape=jax.ShapeDtypeStruct((T, D), jnp.float32),
        scratch_shapes=[
            pltpu.VMEM((SL, D), jnp.float32),
            pltpu.VMEM((T, D), jnp.float32),
            pltpu.VMEM((T, D), jnp.float32),
            pltpu.VMEM((T, D), jnp.float32),
        ],
    )(ctx4, P2, Wv4, bv4, WoR, x, bo, valid, g1, be1,
      W1a, b1a, W2a, b2a, W1b, b1b, W2b, b2b)


def kernel(dec_output, tgt_mask, mem_attn_out, enc_out_mem, tgt_emb_mem,
           tgt_mask_mem, Wq, bq, Wk, bk, Wv, bv, Wo, bo, g0, be0, g1, be1,
           W1a, b1a, W2a, b2a, W1b, b1b, W2b, b2b):
    b, l_tar, d = dec_output.shape

    dec2 = dec_output.reshape(T, D)
    mem2 = mem_attn_out.reshape(T, M + 1)
    mem2T = mem2.T
    row = lambda v: v.reshape(1, -1)

    x, qpad, valid, gent, P2 = _prep(dec2, mem2, mem2T, row(g0), row(be0),
                                     Wq, row(bq))

    qpadT = qpad.reshape(SL, N_HEAD, DH).transpose(1, 0, 2)
    WkR = Wk.reshape(D, N_HEAD, DH).transpose(1, 0, 2)
    Mpad = _mfold(qpadT, WkR)
    Mg5 = Mpad.reshape(NG, GH, D // 128, 128)

    biasT = jnp.where(tgt_mask_mem, 0.0, -1e9).astype(jnp.float32)[:, None, :]
    ctxg = _attend(gent.reshape(NG), Mg5, enc_out_mem, tgt_emb_mem, biasT)
    ctx4 = ctxg.reshape(SL, N_HEAD, D // 128, 128)

    WvR = Wv.reshape(D, N_HEAD, DH).transpose(1, 0, 2)
    WoR = Wo.reshape(N_HEAD, DH, D)
    out = _tail(ctx4, P2, WvR,
                bv.reshape(N_HEAD, 1, DH), WoR, x, row(bo),
                valid, row(g1), row(be1), W1a, row(b1a), W2a, row(b2a),
                W1b, row(b1b), W2b, row(b2b))
    return out.reshape(b, l_tar, d)
